# baseline (speedup 1.0000x reference)
import jax
import jax.numpy as jnp
from jax.experimental import pallas as pl
from jax.experimental.pallas import tpu as pltpu

ALPHA = 0.25
W_CLASS = 2.0
W_BBOX = 5.0
W_GIOU = 2.0
EPS_LOG = 1e-8
EPS_DIV = 1e-6

_CHUNK = 64


def _cost_kernel(logits_ref, pb_ref, tb_ref, tid_ref, out_ref):
    q = out_ref.shape[1]
    t = out_ref.shape[2]

    idm1 = tid_ref[0:1, :] - 1
    cxt = tb_ref[0:1, :]
    cyt = tb_ref[1:2, :]
    wt = tb_ref[2:3, :]
    ht = tb_ref[3:4, :]
    x0t = cxt - 0.5 * wt
    y0t = cyt - 0.5 * ht
    x1t = cxt + 0.5 * wt
    y1t = cyt + 0.5 * ht
    areat = (x1t - x0t) * (y1t - y0t)

    def do_chunk(rows, m):
        s = jax.nn.sigmoid(logits_ref[0, rows, :])
        one_m = 1.0 - s
        neg = (1.0 - ALPHA) * (s * s) * (-jnp.log(one_m + EPS_LOG))
        pos = ALPHA * (one_m * one_m) * (-jnp.log(s + EPS_LOG))
        diff = W_CLASS * (pos - neg)

        idx = jnp.broadcast_to(idm1, (m, t))
        cost_class = jnp.take_along_axis(diff, idx, axis=1)

        cxq = pb_ref[0, rows, 0:1]
        cyq = pb_ref[0, rows, 1:2]
        wq = pb_ref[0, rows, 2:3]
        hq = pb_ref[0, rows, 3:4]

        cost_bbox = (jnp.abs(cxq - cxt) + jnp.abs(cyq - cyt)
                     + jnp.abs(wq - wt) + jnp.abs(hq - ht))

        x0q = cxq - 0.5 * wq
        y0q = cyq - 0.5 * hq
        x1q = cxq + 0.5 * wq
        y1q = cyq + 0.5 * hq
        areaq = (x1q - x0q) * (y1q - y0q)

        xlo = jnp.maximum(x0q, x0t)
        xhi = jnp.minimum(x1q, x1t)
        ylo = jnp.maximum(y0q, y0t)
        yhi = jnp.minimum(y1q, y1t)
        iw = jnp.maximum(xhi - xlo, 0.0)
        ih = jnp.maximum(yhi - ylo, 0.0)
        inter = iw * ih
        union = areaq + areat - inter
        iou = inter / jnp.maximum(union, EPS_DIV)
        ew = jnp.maximum(x1q, x1t) - jnp.minimum(x0q, x0t)
        eh = jnp.maximum(y1q, y1t) - jnp.minimum(y0q, y0t)
        encl = ew * eh
        giou = iou - (encl - union) / jnp.maximum(encl, EPS_DIV)

        out_ref[0, rows, :] = (cost_class + W_BBOX * cost_bbox
                               - W_GIOU * giou)

    n_full = q // _CHUNK

    def body(i, carry):
        a = pl.multiple_of(i * _CHUNK, _CHUNK)
        do_chunk(pl.ds(a, _CHUNK), _CHUNK)
        return carry

    jax.lax.fori_loop(0, n_full, body, 0)
    if q % _CHUNK:
        do_chunk(slice(n_full * _CHUNK, q), q - n_full * _CHUNK)


def kernel(pred_logits, pred_boxes, tgt_ids, tgt_boxes):
    B, Q, C = pred_logits.shape
    T = tgt_ids.shape[0]

    logits = jnp.pad(pred_logits, ((0, 0), (0, 0), (0, 128 - C)))
    tb = jnp.pad(tgt_boxes.T, ((0, 4), (0, 0)))
    tid = tgt_ids.astype(jnp.int32).reshape(1, T)

    return pl.pallas_call(
        _cost_kernel,
        out_shape=jax.ShapeDtypeStruct((B, Q, T), jnp.float32),
        grid=(B,),
        in_specs=[
            pl.BlockSpec((1, Q, 128), lambda i: (i, 0, 0)),
            pl.BlockSpec((1, Q, 4), lambda i: (i, 0, 0)),
            pl.BlockSpec((8, T), lambda i: (0, 0)),
            pl.BlockSpec((1, T), lambda i: (0, 0)),
        ],
        out_specs=pl.BlockSpec((1, Q, T), lambda i: (i, 0, 0)),
        compiler_params=pltpu.CompilerParams(
            dimension_semantics=("arbitrary",),
            vmem_limit_bytes=56 * 1024 * 1024,
        ),
        name="hungarian_cost_matrix",
    )(logits, pred_boxes, tb, tid)


# speedup vs baseline: 1.4869x; 1.4869x over previous
import jax
import jax.numpy as jnp
from jax.experimental import pallas as pl
from jax.experimental.pallas import tpu as pltpu

ALPHA = 0.25
W_CLASS = 2.0
W_BBOX = 5.0
W_GIOU = 2.0
EPS_LOG = 1e-8
EPS_DIV = 1e-6

_QCHUNK = 64
_TSLICE = 128


def _cost_kernel(logits_ref, pb_ref, tb_ref, tid_ref, out_ref):
    q = out_ref.shape[1]
    t = out_ref.shape[2]

    idm1 = tid_ref[0:1, :] - 1
    cxt = tb_ref[0:1, :]
    cyt = tb_ref[1:2, :]
    wt = tb_ref[2:3, :]
    ht = tb_ref[3:4, :]
    x0t = cxt - 0.5 * wt
    y0t = cyt - 0.5 * ht
    x1t = cxt + 0.5 * wt
    y1t = cyt + 0.5 * ht
    areat = (x1t - x0t) * (y1t - y0t)

    def do_chunk(rows, m):
        s = jax.nn.sigmoid(logits_ref[0, rows, :])
        one_m = 1.0 - s
        neg = (1.0 - ALPHA) * (s * s) * (-jnp.log(one_m + EPS_LOG))
        pos = ALPHA * (one_m * one_m) * (-jnp.log(s + EPS_LOG))
        diff = W_CLASS * (pos - neg)

        cxq = pb_ref[0, rows, 0:1]
        cyq = pb_ref[0, rows, 1:2]
        wq = pb_ref[0, rows, 2:3]
        hq = pb_ref[0, rows, 3:4]
        x0q = cxq - 0.5 * wq
        y0q = cyq - 0.5 * hq
        x1q = cxq + 0.5 * wq
        y1q = cyq + 0.5 * hq
        areaq = (x1q - x0q) * (y1q - y0q)

        for c0 in range(0, t, _TSLICE):
            c1 = min(c0 + _TSLICE, t)
            n = c1 - c0
            cols = slice(c0, c1)

            idx = jnp.broadcast_to(idm1[:, cols], (m, n))
            cost_class = jnp.take_along_axis(diff, idx, axis=1)

            cost_bbox = (jnp.abs(cxq - cxt[:, cols])
                         + jnp.abs(cyq - cyt[:, cols])
                         + jnp.abs(wq - wt[:, cols])
                         + jnp.abs(hq - ht[:, cols]))

            x0 = x0t[:, cols]
            y0 = y0t[:, cols]
            x1 = x1t[:, cols]
            y1 = y1t[:, cols]

            iw = jnp.maximum(jnp.minimum(x1q, x1) - jnp.maximum(x0q, x0), 0.0)
            ih = jnp.maximum(jnp.minimum(y1q, y1) - jnp.maximum(y0q, y0), 0.0)
            inter = iw * ih
            union = areaq + areat[:, cols] - inter
            iou = inter / jnp.maximum(union, EPS_DIV)
            ew = jnp.maximum(x1q, x1) - jnp.minimum(x0q, x0)
            eh = jnp.maximum(y1q, y1) - jnp.minimum(y0q, y0)
            encl = ew * eh
            giou = iou - (encl - union) / jnp.maximum(encl, EPS_DIV)

            out_ref[0, rows, cols] = (cost_class + W_BBOX * cost_bbox
                                      - W_GIOU * giou)

    n_full = q // _QCHUNK

    def body(i, carry):
        a = pl.multiple_of(i * _QCHUNK, _QCHUNK)
        do_chunk(pl.ds(a, _QCHUNK), _QCHUNK)
        return carry

    jax.lax.fori_loop(0, n_full, body, 0)
    if q % _QCHUNK:
        do_chunk(slice(n_full * _QCHUNK, q), q - n_full * _QCHUNK)


def kernel(pred_logits, pred_boxes, tgt_ids, tgt_boxes):
    B, Q, C = pred_logits.shape
    T = tgt_ids.shape[0]

    logits = jnp.pad(pred_logits, ((0, 0), (0, 0), (0, 128 - C)))
    tb = jnp.pad(tgt_boxes.T, ((0, 4), (0, 0)))
    tid = tgt_ids.astype(jnp.int32).reshape(1, T)

    return pl.pallas_call(
        _cost_kernel,
        out_shape=jax.ShapeDtypeStruct((B, Q, T), jnp.float32),
        grid=(B,),
        in_specs=[
            pl.BlockSpec((1, Q, 128), lambda i: (i, 0, 0)),
            pl.BlockSpec((1, Q, 4), lambda i: (i, 0, 0)),
            pl.BlockSpec((8, T), lambda i: (0, 0)),
            pl.BlockSpec((1, T), lambda i: (0, 0)),
        ],
        out_specs=pl.BlockSpec((1, Q, T), lambda i: (i, 0, 0)),
        compiler_params=pltpu.CompilerParams(
            dimension_semantics=("arbitrary",),
            vmem_limit_bytes=56 * 1024 * 1024,
        ),
        name="hungarian_cost_matrix",
    )(logits, pred_boxes, tb, tid)


# speedup vs baseline: 1.5105x; 1.0159x over previous
import jax
import jax.numpy as jnp
from jax.experimental import pallas as pl
from jax.experimental.pallas import tpu as pltpu

ALPHA = 0.25
W_CLASS = 2.0
W_BBOX = 5.0
W_GIOU = 2.0
EPS_LOG = 1e-8
EPS_DIV = 1e-6

_QCHUNK = 128
_TSLICE = 128


def _cost_kernel(logits_ref, pb_ref, tb_ref, tid_ref, out_ref):
    q = out_ref.shape[1]
    t = out_ref.shape[2]

    idm1 = tid_ref[0:1, :] - 1
    cxt = tb_ref[0:1, :]
    cyt = tb_ref[1:2, :]
    wt = tb_ref[2:3, :]
    ht = tb_ref[3:4, :]
    x0t = cxt - 0.5 * wt
    y0t = cyt - 0.5 * ht
    x1t = cxt + 0.5 * wt
    y1t = cyt + 0.5 * ht
    areat = (x1t - x0t) * (y1t - y0t)

    def do_chunk(rows, m):
        s = jax.nn.sigmoid(logits_ref[0, rows, :])
        one_m = 1.0 - s
        neg = (1.0 - ALPHA) * (s * s) * (-jnp.log(one_m + EPS_LOG))
        pos = ALPHA * (one_m * one_m) * (-jnp.log(s + EPS_LOG))
        diff = W_CLASS * (pos - neg)

        cxq = pb_ref[0, rows, 0:1]
        cyq = pb_ref[0, rows, 1:2]
        wq = pb_ref[0, rows, 2:3]
        hq = pb_ref[0, rows, 3:4]
        x0q = cxq - 0.5 * wq
        y0q = cyq - 0.5 * hq
        x1q = cxq + 0.5 * wq
        y1q = cyq + 0.5 * hq
        areaq = (x1q - x0q) * (y1q - y0q)

        for c0 in range(0, t, _TSLICE):
            c1 = min(c0 + _TSLICE, t)
            n = c1 - c0
            cols = slice(c0, c1)

            idx = jnp.broadcast_to(idm1[:, cols], (m, n))
            cost_class = jnp.take_along_axis(diff, idx, axis=1)

            cost_bbox = (jnp.abs(cxq - cxt[:, cols])
                         + jnp.abs(cyq - cyt[:, cols])
                         + jnp.abs(wq - wt[:, cols])
                         + jnp.abs(hq - ht[:, cols]))

            x0 = x0t[:, cols]
            y0 = y0t[:, cols]
            x1 = x1t[:, cols]
            y1 = y1t[:, cols]

            iw = jnp.maximum(jnp.minimum(x1q, x1) - jnp.maximum(x0q, x0), 0.0)
            ih = jnp.maximum(jnp.minimum(y1q, y1) - jnp.maximum(y0q, y0), 0.0)
            inter = iw * ih
            union = areaq + areat[:, cols] - inter
            iou = inter / jnp.maximum(union, EPS_DIV)
            ew = jnp.maximum(x1q, x1) - jnp.minimum(x0q, x0)
            eh = jnp.maximum(y1q, y1) - jnp.minimum(y0q, y0)
            encl = ew * eh
            giou = iou - (encl - union) / jnp.maximum(encl, EPS_DIV)

            out_ref[0, rows, cols] = (cost_class + W_BBOX * cost_bbox
                                      - W_GIOU * giou)

    n_full = q // _QCHUNK

    def body(i, carry):
        a = pl.multiple_of(i * _QCHUNK, _QCHUNK)
        do_chunk(pl.ds(a, _QCHUNK), _QCHUNK)
        return carry

    jax.lax.fori_loop(0, n_full, body, 0)
    if q % _QCHUNK:
        do_chunk(slice(n_full * _QCHUNK, q), q - n_full * _QCHUNK)


def kernel(pred_logits, pred_boxes, tgt_ids, tgt_boxes):
    B, Q, C = pred_logits.shape
    T = tgt_ids.shape[0]

    logits = jnp.pad(pred_logits, ((0, 0), (0, 0), (0, 128 - C)))
    tb = jnp.pad(tgt_boxes.T, ((0, 4), (0, 0)))
    tid = tgt_ids.astype(jnp.int32).reshape(1, T)

    return pl.pallas_call(
        _cost_kernel,
        out_shape=jax.ShapeDtypeStruct((B, Q, T), jnp.float32),
        grid=(B,),
        in_specs=[
            pl.BlockSpec((1, Q, 128), lambda i: (i, 0, 0)),
            pl.BlockSpec((1, Q, 4), lambda i: (i, 0, 0)),
            pl.BlockSpec((8, T), lambda i: (0, 0)),
            pl.BlockSpec((1, T), lambda i: (0, 0)),
        ],
        out_specs=pl.BlockSpec((1, Q, T), lambda i: (i, 0, 0)),
        compiler_params=pltpu.CompilerParams(
            dimension_semantics=("arbitrary",),
            vmem_limit_bytes=56 * 1024 * 1024,
        ),
        name="hungarian_cost_matrix",
    )(logits, pred_boxes, tb, tid)
